# baseline (speedup 1.0000x reference)
import jax
import jax.numpy as jnp
import numpy as np
from jax import lax
from jax.experimental import pallas as pl
from jax.experimental.pallas import tpu as pltpu
from jax.experimental.pallas import tpu_sc as plsc

BATCH = 4
MODEL_DIM = 1024
MAX_LEN = 2048

NC = 2
NS = 16
LANES = 16
NW = NC * NS

B_TOTAL = BATCH * MAX_LEN
T_PER_W = MAX_LEN // NW
CHUNK = 16
QR = 8
QPC = CHUNK // QR
N_TC = T_PER_W // CHUNK
N_CHUNKS = N_TC * BATCH
NB = 5
GLEAD = 3
NQ = MAX_LEN // QR
NQL = 4
NQH = NQ // NQL

_VR = MODEL_DIM // LANES

_OFF_AH1 = 0
_OFF_AH2 = NQH * MODEL_DIM
_OFF_ALB = 2 * NQH * MODEL_DIM
_ALB_LEN = (2 * NQL + 2 * QR) * MODEL_DIM
_AL2_OFF = NQL * MODEL_DIM
_B1_OFF = 2 * NQL * MODEL_DIM
_B2_OFF = 2 * NQL * MODEL_DIM + QR * MODEL_DIM


def _aux_tables_np():
    i = np.arange(MODEL_DIM, dtype=np.float64)
    w = 1.0 / (10000.0 ** (2.0 * i / MODEL_DIM))
    even = (np.arange(MODEL_DIM) % 2) == 0
    thh = QR * NQL * np.arange(NQH, dtype=np.float64).reshape(-1, 1) * w
    ah1 = np.where(even, np.sin(thh), np.cos(thh)).astype(np.float32)
    ah2 = np.where(even, np.cos(thh), -np.sin(thh)).astype(np.float32)
    thl = QR * np.arange(NQL, dtype=np.float64).reshape(-1, 1) * w
    al1 = np.cos(thl).astype(np.float32)
    al2 = np.sin(thl).astype(np.float32)
    rw = np.arange(QR, dtype=np.float64).reshape(-1, 1) * w
    b1 = np.cos(rw).astype(np.float32)
    b2 = np.sin(rw).astype(np.float32)
    return np.concatenate(
        [x.reshape(-1) for x in (ah1, ah2, al1, al2, b1, b2)]
    )


_AUX = _aux_tables_np()


def _sc_body(
    table_hbm, idx_hbm, aux_hbm, out_hbm,
    idx_v, rows_all, posrec, b_v, a_all, gsem, psem, asem, bsem, isem,
):
    wid = lax.axis_index("s") * NC + lax.axis_index("c")
    t0 = wid * T_PER_W

    bd = pltpu.async_copy(
        aux_hbm.at[pl.ds(_OFF_ALB, _ALB_LEN)], b_v, bsem
    )

    def issue_a(tc):
        qh = (wid * N_TC + tc) * QPC // NQL
        buf = lax.rem(tc, 2)
        pltpu.async_copy(
            aux_hbm.at[pl.ds(_OFF_AH1 + qh * MODEL_DIM, MODEL_DIM)],
            a_all.at[buf].at[pl.ds(0, MODEL_DIM)],
            asem.at[buf],
        )
        pltpu.async_copy(
            aux_hbm.at[pl.ds(_OFF_AH2 + qh * MODEL_DIM, MODEL_DIM)],
            a_all.at[buf].at[pl.ds(MODEL_DIM, MODEL_DIM)],
            asem.at[buf],
        )

    issue_a(0)
    issue_a(1)

    idx_descs = [
        pltpu.async_copy(
            idx_hbm.at[b, pl.ds(t0, T_PER_W)],
            idx_v.at[pl.ds(b * T_PER_W, T_PER_W)],
            isem,
        )
        for b in range(BATCH)
    ]
    for d in idx_descs:
        d.wait()

    def issue_gather(n):
        tc = n // BATCH
        b = lax.rem(n, BATCH)
        jn = lax.rem(n, NB)
        pltpu.async_copy(
            table_hbm.at[idx_v.at[pl.ds(b * T_PER_W + tc * CHUNK, CHUNK)]],
            rows_all.at[jn],
            gsem.at[jn],
        )

    def wait_dma(sem, vmem_ref):
        pltpu.make_async_copy(
            table_hbm.at[pl.ds(0, CHUNK)], vmem_ref, sem
        ).wait()

    for n in range(GLEAD):
        issue_gather(n)

    def chunk_body(c, _):
        j = lax.rem(c, NB)
        tc = c // BATCH
        b = lax.rem(c, BATCH)

        n = c + GLEAD
        jn = lax.rem(n, NB)

        @pl.when(n < N_CHUNKS)
        def _():
            @pl.when(n >= NB)
            def _():
                wait_dma(psem.at[jn], rows_all.at[jn])

            issue_gather(n)

        @pl.when(b == 0)
        def _():
            buf = lax.rem(tc, 2)
            a_tc = a_all.at[buf]
            pltpu.make_async_copy(
                aux_hbm.at[pl.ds(0, 2 * MODEL_DIM)], a_tc, asem.at[buf]
            ).wait()
            q0 = (wid * N_TC + tc) * QPC
            ql0 = lax.rem(q0, NQL)

            @plsc.parallel_loop(0, _VR, unroll=1)
            def gen_body(jc):
                off = pl.multiple_of(jc << 4, LANES)
                ah1 = a_tc[pl.ds(off, LANES)]
                ah2 = a_tc[pl.ds(MODEL_DIM + off, LANES)]
                al1q0 = b_v[pl.ds(ql0 * MODEL_DIM + off, LANES)]
                al2q0 = b_v[pl.ds(_AL2_OFF + ql0 * MODEL_DIM + off, LANES)]
                al1q1 = b_v[pl.ds((ql0 + 1) * MODEL_DIM + off, LANES)]
                al2q1 = b_v[pl.ds(_AL2_OFF + (ql0 + 1) * MODEL_DIM + off, LANES)]
                a1q0 = ah1 * al1q0 + ah2 * al2q0
                a2q0 = ah2 * al1q0 - ah1 * al2q0
                a1q1 = ah1 * al1q1 + ah2 * al2q1
                a2q1 = ah2 * al1q1 - ah1 * al2q1
                for r in range(QR):
                    b1r = b_v[pl.ds(_B1_OFF + r * MODEL_DIM + off, LANES)]
                    b2r = b_v[pl.ds(_B2_OFF + r * MODEL_DIM + off, LANES)]
                    posrec[r, pl.ds(off, LANES)] = a1q0 * b1r + a2q0 * b2r
                    posrec[QR + r, pl.ds(off, LANES)] = a1q1 * b1r + a2q1 * b2r

            @pl.when(tc < N_TC - 2)
            def _():
                issue_a(tc + 2)

        wait_dma(gsem.at[j], rows_all.at[j])
        rows_j = rows_all.at[j]

        @plsc.parallel_loop(0, CHUNK * _VR, unroll=4)
        def add_body(i):
            r = i >> 6
            off = pl.multiple_of((i & (_VR - 1)) << 4, LANES)
            rows_j[r, pl.ds(off, LANES)] = (
                rows_j[r, pl.ds(off, LANES)] + posrec[r, pl.ds(off, LANES)]
            )

        pltpu.async_copy(
            rows_j,
            out_hbm.at[pl.ds(b * MAX_LEN + t0 + tc * CHUNK, CHUNK)],
            psem.at[j],
        )
        return 0

    bd.wait()
    lax.fori_loop(0, N_CHUNKS, chunk_body, 0)

    for c in range(N_CHUNKS - NB, N_CHUNKS):
        wait_dma(psem.at[c % NB], rows_all.at[c % NB])


@jax.jit
def _embed(idx, table, aux):
    mesh = plsc.VectorSubcoreMesh(
        core_axis_name="c", subcore_axis_name="s", num_cores=NC, num_subcores=NS
    )
    scratch = [
        pltpu.VMEM((BATCH * T_PER_W,), jnp.int32),
        pltpu.VMEM((NB, CHUNK, MODEL_DIM), jnp.float32),
        pltpu.VMEM((CHUNK, MODEL_DIM), jnp.float32),
        pltpu.VMEM((_ALB_LEN,), jnp.float32),
        pltpu.VMEM((2, 2 * MODEL_DIM), jnp.float32),
        pltpu.SemaphoreType.DMA((NB,)),
        pltpu.SemaphoreType.DMA((NB,)),
        pltpu.SemaphoreType.DMA((2,)),
        pltpu.SemaphoreType.DMA,
        pltpu.SemaphoreType.DMA,
    ]
    fn = pl.kernel(
        _sc_body,
        out_type=jax.ShapeDtypeStruct((B_TOTAL, MODEL_DIM), jnp.float32),
        mesh=mesh,
        scratch_types=scratch,
    )
    return fn(table, idx, aux)


def kernel(x, table):
    idx = x.astype(jnp.int32)
    out = _embed(idx, table, jnp.asarray(_AUX))
    return out.reshape(BATCH, MAX_LEN, MODEL_DIM)


# speedup vs baseline: 1.0030x; 1.0030x over previous
import jax
import jax.numpy as jnp
import numpy as np
from jax import lax
from jax.experimental import pallas as pl
from jax.experimental.pallas import tpu as pltpu
from jax.experimental.pallas import tpu_sc as plsc

BATCH = 4
MODEL_DIM = 1024
MAX_LEN = 2048

NC = 2
NS = 16
LANES = 16
NW = NC * NS

B_TOTAL = BATCH * MAX_LEN
T_PER_W = MAX_LEN // NW
CHUNK = 16
QR = 8
QPC = CHUNK // QR
N_TC = T_PER_W // CHUNK
N_CHUNKS = N_TC * BATCH
NB = 5
GLEAD = 3
NQ = MAX_LEN // QR
NQL = 4
NQH = NQ // NQL

_VR = MODEL_DIM // LANES

_OFF_AH1 = 0
_OFF_AH2 = NQH * MODEL_DIM
_OFF_ALB = 2 * NQH * MODEL_DIM
_ALB_LEN = (2 * NQL + 2 * QR) * MODEL_DIM
_AL2_OFF = NQL * MODEL_DIM
_B1_OFF = 2 * NQL * MODEL_DIM
_B2_OFF = 2 * NQL * MODEL_DIM + QR * MODEL_DIM


def _aux_tables_np():
    i = np.arange(MODEL_DIM, dtype=np.float64)
    w = 1.0 / (10000.0 ** (2.0 * i / MODEL_DIM))
    even = (np.arange(MODEL_DIM) % 2) == 0
    thh = QR * NQL * np.arange(NQH, dtype=np.float64).reshape(-1, 1) * w
    ah1 = np.where(even, np.sin(thh), np.cos(thh)).astype(np.float32)
    ah2 = np.where(even, np.cos(thh), -np.sin(thh)).astype(np.float32)
    thl = QR * np.arange(NQL, dtype=np.float64).reshape(-1, 1) * w
    al1 = np.cos(thl).astype(np.float32)
    al2 = np.sin(thl).astype(np.float32)
    rw = np.arange(QR, dtype=np.float64).reshape(-1, 1) * w
    b1 = np.cos(rw).astype(np.float32)
    b2 = np.sin(rw).astype(np.float32)
    return np.concatenate(
        [x.reshape(-1) for x in (ah1, ah2, al1, al2, b1, b2)]
    )


_AUX = _aux_tables_np()


def _sc_body(
    table_hbm, idx_hbm, aux_hbm, out_hbm,
    idx_v, rows_all, posrec, b_v, a_all, gsem, psem, asem, bsem, isem,
):
    wid = lax.axis_index("s") * NC + lax.axis_index("c")
    t0 = wid * T_PER_W

    bd = pltpu.async_copy(
        aux_hbm.at[pl.ds(_OFF_ALB, _ALB_LEN)], b_v, bsem
    )

    def issue_a(tc):
        qh = (wid * N_TC + tc) * QPC // NQL
        buf = lax.rem(tc, 2)
        pltpu.async_copy(
            aux_hbm.at[pl.ds(_OFF_AH1 + qh * MODEL_DIM, MODEL_DIM)],
            a_all.at[buf].at[pl.ds(0, MODEL_DIM)],
            asem.at[buf],
        )
        pltpu.async_copy(
            aux_hbm.at[pl.ds(_OFF_AH2 + qh * MODEL_DIM, MODEL_DIM)],
            a_all.at[buf].at[pl.ds(MODEL_DIM, MODEL_DIM)],
            asem.at[buf],
        )

    issue_a(0)
    issue_a(1)

    idx_descs = [
        pltpu.async_copy(
            idx_hbm.at[b, pl.ds(t0, T_PER_W)],
            idx_v.at[pl.ds(b * T_PER_W, T_PER_W)],
            isem,
        )
        for b in range(BATCH)
    ]
    for d in idx_descs:
        d.wait()

    def issue_gather(n):
        tc = n // BATCH
        b = lax.rem(n, BATCH)
        jn = lax.rem(n, NB)
        pltpu.async_copy(
            table_hbm.at[idx_v.at[pl.ds(b * T_PER_W + tc * CHUNK, CHUNK)]],
            rows_all.at[jn],
            gsem.at[jn],
        )

    def wait_dma(sem, vmem_ref):
        pltpu.make_async_copy(
            table_hbm.at[pl.ds(0, CHUNK)], vmem_ref, sem
        ).wait()

    for n in range(GLEAD):
        issue_gather(n)

    def chunk_body(c, _):
        j = lax.rem(c, NB)
        tc = c // BATCH
        b = lax.rem(c, BATCH)

        n = c + GLEAD
        jn = lax.rem(n, NB)

        @pl.when(n < N_CHUNKS)
        def _():
            @pl.when(n >= NB)
            def _():
                wait_dma(psem.at[jn], rows_all.at[jn])

            issue_gather(n)

        @pl.when(b == 0)
        def _():
            buf = lax.rem(tc, 2)
            a_tc = a_all.at[buf]
            pltpu.make_async_copy(
                aux_hbm.at[pl.ds(0, 2 * MODEL_DIM)], a_tc, asem.at[buf]
            ).wait()
            q0 = (wid * N_TC + tc) * QPC
            ql0 = lax.rem(q0, NQL)

            @plsc.parallel_loop(0, _VR, unroll=1)
            def gen_body(jc):
                off = pl.multiple_of(jc << 4, LANES)
                ah1 = a_tc[pl.ds(off, LANES)]
                ah2 = a_tc[pl.ds(MODEL_DIM + off, LANES)]
                al1q0 = b_v[pl.ds(ql0 * MODEL_DIM + off, LANES)]
                al2q0 = b_v[pl.ds(_AL2_OFF + ql0 * MODEL_DIM + off, LANES)]
                al1q1 = b_v[pl.ds((ql0 + 1) * MODEL_DIM + off, LANES)]
                al2q1 = b_v[pl.ds(_AL2_OFF + (ql0 + 1) * MODEL_DIM + off, LANES)]
                a1q0 = ah1 * al1q0 + ah2 * al2q0
                a2q0 = ah2 * al1q0 - ah1 * al2q0
                a1q1 = ah1 * al1q1 + ah2 * al2q1
                a2q1 = ah2 * al1q1 - ah1 * al2q1
                for r in range(QR):
                    b1r = b_v[pl.ds(_B1_OFF + r * MODEL_DIM + off, LANES)]
                    b2r = b_v[pl.ds(_B2_OFF + r * MODEL_DIM + off, LANES)]
                    posrec[r, pl.ds(off, LANES)] = a1q0 * b1r + a2q0 * b2r
                    posrec[QR + r, pl.ds(off, LANES)] = a1q1 * b1r + a2q1 * b2r

            @pl.when(tc < N_TC - 2)
            def _():
                issue_a(tc + 2)

        wait_dma(gsem.at[j], rows_all.at[j])
        rows_j = rows_all.at[j]

        @plsc.parallel_loop(0, CHUNK * _VR, unroll=8)
        def add_body(i):
            r = i >> 6
            off = pl.multiple_of((i & (_VR - 1)) << 4, LANES)
            rows_j[r, pl.ds(off, LANES)] = (
                rows_j[r, pl.ds(off, LANES)] + posrec[r, pl.ds(off, LANES)]
            )

        pltpu.async_copy(
            rows_j,
            out_hbm.at[pl.ds(b * MAX_LEN + t0 + tc * CHUNK, CHUNK)],
            psem.at[j],
        )
        return 0

    bd.wait()
    lax.fori_loop(0, N_CHUNKS, chunk_body, 0)

    for c in range(N_CHUNKS - NB, N_CHUNKS):
        wait_dma(psem.at[c % NB], rows_all.at[c % NB])


@jax.jit
def _embed(idx, table, aux):
    mesh = plsc.VectorSubcoreMesh(
        core_axis_name="c", subcore_axis_name="s", num_cores=NC, num_subcores=NS
    )
    scratch = [
        pltpu.VMEM((BATCH * T_PER_W,), jnp.int32),
        pltpu.VMEM((NB, CHUNK, MODEL_DIM), jnp.float32),
        pltpu.VMEM((CHUNK, MODEL_DIM), jnp.float32),
        pltpu.VMEM((_ALB_LEN,), jnp.float32),
        pltpu.VMEM((2, 2 * MODEL_DIM), jnp.float32),
        pltpu.SemaphoreType.DMA((NB,)),
        pltpu.SemaphoreType.DMA((NB,)),
        pltpu.SemaphoreType.DMA((2,)),
        pltpu.SemaphoreType.DMA,
        pltpu.SemaphoreType.DMA,
    ]
    fn = pl.kernel(
        _sc_body,
        out_type=jax.ShapeDtypeStruct((B_TOTAL, MODEL_DIM), jnp.float32),
        mesh=mesh,
        scratch_types=scratch,
    )
    return fn(table, idx, aux)


def kernel(x, table):
    idx = x.astype(jnp.int32)
    out = _embed(idx, table, jnp.asarray(_AUX))
    return out.reshape(BATCH, MAX_LEN, MODEL_DIM)
